# Initial kernel scaffold; baseline (speedup 1.0000x reference)
import jax
import jax.numpy as jnp
from jax.experimental import pallas as pl


def kernel(x, edge_index, W, att_src, att_dst, bias, lin_w, lin_b):
    raise NotImplementedError("write your pallas kernel here")


# speedup vs baseline: 1.0935x; 1.0935x over previous
import functools

import jax
import jax.numpy as jnp
from jax.experimental import pallas as pl
from jax.experimental.pallas import tpu as pltpu

N = 10000
E = 160000
D = 256
H = 8
C = 256
NUM_CLASSES = 40

_BN = 2000


def _mm_kernel(x_ref, w_ref, o_ref):
    o_ref[...] = jnp.dot(x_ref[...], w_ref[...],
                         preferred_element_type=jnp.float32)


def _matmul(x, w):
    n, d = x.shape
    d2, m = w.shape
    return pl.pallas_call(
        _mm_kernel,
        grid=(n // _BN,),
        in_specs=[
            pl.BlockSpec((_BN, d), lambda i: (i, 0)),
            pl.BlockSpec((d2, m), lambda i: (0, 0)),
        ],
        out_specs=pl.BlockSpec((_BN, m), lambda i: (i, 0)),
        out_shape=jax.ShapeDtypeStruct((n, m), jnp.float32),
    )(x, w)


def _head_kernel(agg_ref, w_ref, bias_ref, lw_ref, lb_ref, o_ref):
    agg = agg_ref[...]
    w = w_ref[...]
    hs = []
    for h in range(H):
        hs.append(jnp.dot(agg[:, h * D:(h + 1) * D], w[:, h * C:(h + 1) * C],
                          preferred_element_type=jnp.float32))
    hcat = jnp.concatenate(hs, axis=1) + bias_ref[...]
    hcat = jnp.maximum(hcat, 0.0)
    o_ref[...] = jnp.dot(hcat, lw_ref[...],
                         preferred_element_type=jnp.float32) + lb_ref[...]


def _head(agg, W, bias, lin_w, lin_b):
    return pl.pallas_call(
        _head_kernel,
        grid=(N // _BN,),
        in_specs=[
            pl.BlockSpec((_BN, H * D), lambda i: (i, 0)),
            pl.BlockSpec((D, H * C), lambda i: (0, 0)),
            pl.BlockSpec((1, H * C), lambda i: (0, 0)),
            pl.BlockSpec((H * C, NUM_CLASSES), lambda i: (0, 0)),
            pl.BlockSpec((1, NUM_CLASSES), lambda i: (0, 0)),
        ],
        out_specs=pl.BlockSpec((_BN, NUM_CLASSES), lambda i: (i, 0)),
        out_shape=jax.ShapeDtypeStruct((N, NUM_CLASSES), jnp.float32),
    )(agg, W, bias.reshape(1, -1), lin_w, lin_b.reshape(1, -1))


def kernel(x, edge_index, W, att_src, att_dst, bias, lin_w, lin_b):
    Wr = W.reshape(D, H, C)
    M_src = jnp.einsum("dhc,hc->dh", Wr, att_src)
    M_dst = jnp.einsum("dhc,hc->dh", Wr, att_dst)
    a = _matmul(x, jnp.concatenate([M_src, M_dst], axis=1))
    a_src, a_dst = a[:, :H], a[:, H:]

    src = edge_index[0]
    dst = edge_index[1]
    alpha = a_src[src] + a_dst[dst]
    alpha = jnp.where(alpha > 0, alpha, 0.2 * alpha)
    ealpha = jnp.exp(alpha)
    denom = jax.ops.segment_sum(ealpha, dst, num_segments=N)
    w_e = ealpha / (denom[dst] + 1e-16)

    msg = x[src][:, None, :] * w_e[:, :, None]
    agg = jax.ops.segment_sum(msg, dst, num_segments=N)
    agg = agg.reshape(N, H * D)

    return _head(agg, W, bias, lin_w, lin_b)
